# baseline (device time: 112574 ns/iter reference)
import jax
import jax.numpy as jnp
from jax import lax
from jax.experimental import pallas as pl
from jax.experimental.pallas import tpu as pltpu

N_DEV = 8
N_TILE = 1024
NBUF = 3


def kernel(x, w_mat, scale_x, scale_w):
    m_total, k_per = x.shape
    k_total, n = w_mat.shape
    m_per = m_total // N_DEV
    n_panels = n // N_TILE
    n_chunks = n_panels * N_DEV

    def body(x_ref, w_hbm, sx_ref, sw_ref, out_ref,
             comm_ref, stage_ref, a_ref, wf32_ref, w8_ref,
             send_sems, recv_sems, wsems):
        my = lax.axis_index("i")

        barrier_sem = pltpu.get_barrier_semaphore()
        for p in range(N_DEV):
            pl.semaphore_signal(
                barrier_sem, inc=1,
                device_id=(p,), device_id_type=pl.DeviceIdType.MESH,
            )
        pl.semaphore_wait(barrier_sem, N_DEV)

        def send_desc(d):
            return pltpu.make_async_remote_copy(
                src_ref=stage_ref.at[d],
                dst_ref=comm_ref.at[my],
                send_sem=send_sems.at[d],
                recv_sem=recv_sems.at[my],
                device_id=(d,),
                device_id_type=pl.DeviceIdType.MESH,
            )

        def recv_desc(src):
            return pltpu.make_async_remote_copy(
                src_ref=comm_ref.at[src],
                dst_ref=comm_ref.at[src],
                send_sem=send_sems.at[src],
                recv_sem=recv_sems.at[src],
                device_id=(src,),
                device_id_type=pl.DeviceIdType.MESH,
            )

        def wcopy(k):
            s, p = k % N_DEV, k // N_DEV
            return pltpu.make_async_copy(
                w_hbm.at[pl.ds(s * k_per, k_per), pl.ds(p * N_TILE, N_TILE)],
                wf32_ref.at[k % NBUF],
                wsems.at[k % 4],
            )

        for d in range(N_DEV):
            stage_ref[d] = x_ref[pl.ds(d * m_per, m_per), :].astype(
                jnp.float8_e5m2
            )
        for d in range(N_DEV):
            send_desc(d).start()

        scale = sx_ref[0] * sw_ref[0]

        wcopy(0).start()
        wcopy(1).start()
        for k in range(n_chunks):
            if k + 2 < n_chunks:
                wcopy(k + 2).start()
            wcopy(k).wait()
            s, p = k % N_DEV, k // N_DEV
            w8_ref[pl.ds(s * k_per, k_per), :] = wf32_ref[k % NBUF].astype(
                jnp.bfloat16
            )
            if s == N_DEV - 1:
                if p == 0:
                    for src in range(N_DEV):
                        recv_desc(src).wait_recv()
                        a_ref[:, pl.ds(src * k_per, k_per)] = comm_ref[
                            src].astype(jnp.bfloat16)
                part = jnp.dot(
                    a_ref[...], w8_ref[...],
                    preferred_element_type=jnp.float32,
                )
                out_ref[:, pl.ds(p * N_TILE, N_TILE)] = part * scale

        for d in range(N_DEV):
            send_desc(d).wait_send()

    return pl.pallas_call(
        body,
        out_shape=jax.ShapeDtypeStruct((m_per, n), jnp.float32),
        in_specs=[
            pl.BlockSpec(memory_space=pltpu.VMEM),
            pl.BlockSpec(memory_space=pltpu.MemorySpace.HBM),
            pl.BlockSpec(memory_space=pltpu.SMEM),
            pl.BlockSpec(memory_space=pltpu.SMEM),
        ],
        out_specs=pl.BlockSpec(memory_space=pltpu.VMEM),
        scratch_shapes=[
            pltpu.VMEM((N_DEV, m_per, k_per), jnp.float8_e5m2),
            pltpu.VMEM((N_DEV, m_per, k_per), jnp.float8_e5m2),
            pltpu.VMEM((m_per, k_total), jnp.bfloat16),
            pltpu.VMEM((NBUF, k_per, N_TILE), jnp.float32),
            pltpu.VMEM((k_total, N_TILE), jnp.bfloat16),
            pltpu.SemaphoreType.DMA((N_DEV,)),
            pltpu.SemaphoreType.DMA((N_DEV,)),
            pltpu.SemaphoreType.DMA((4,)),
        ],
        compiler_params=pltpu.CompilerParams(
            collective_id=0,
            vmem_limit_bytes=100 * 1024 * 1024,
        ),
    )(x, w_mat, scale_x, scale_w)


# device time: 82972 ns/iter; 1.3568x vs baseline; 1.3568x over previous
import jax
import jax.numpy as jnp
from jax import lax
from jax.experimental import pallas as pl
from jax.experimental.pallas import tpu as pltpu

N_DEV = 8
N_TILE = 2048
NBUF = 3


def kernel(x, w_mat, scale_x, scale_w):
    m_total, k_per = x.shape
    k_total, n = w_mat.shape
    m_per = m_total // N_DEV
    n_panels = n // N_TILE
    n_chunks = n_panels * N_DEV

    def body(x_ref, w_hbm, sx_ref, sw_ref, out_ref,
             comm_ref, stage_ref, a_ref, wf32_ref, w8_ref,
             send_sems, recv_sems, wsems):
        my = lax.axis_index("i")

        barrier_sem = pltpu.get_barrier_semaphore()
        for p in range(N_DEV):
            pl.semaphore_signal(
                barrier_sem, inc=1,
                device_id=(p,), device_id_type=pl.DeviceIdType.MESH,
            )

        def send_desc(d):
            return pltpu.make_async_remote_copy(
                src_ref=stage_ref.at[d],
                dst_ref=comm_ref.at[my],
                send_sem=send_sems.at[d],
                recv_sem=recv_sems.at[my],
                device_id=(d,),
                device_id_type=pl.DeviceIdType.MESH,
            )

        def recv_desc(src):
            return pltpu.make_async_remote_copy(
                src_ref=comm_ref.at[src],
                dst_ref=comm_ref.at[src],
                send_sem=send_sems.at[src],
                recv_sem=recv_sems.at[src],
                device_id=(src,),
                device_id_type=pl.DeviceIdType.MESH,
            )

        def wcopy(k):
            s, p = k % N_DEV, k // N_DEV
            return pltpu.make_async_copy(
                w_hbm.at[pl.ds(s * k_per, k_per), pl.ds(p * N_TILE, N_TILE)],
                wf32_ref.at[k % NBUF],
                wsems.at[k % 4],
            )

        for k in range(NBUF):
            wcopy(k).start()
        for d in range(N_DEV):
            stage_ref[d] = x_ref[pl.ds(d * m_per, m_per), :].astype(
                jnp.float8_e5m2
            )
        pl.semaphore_wait(barrier_sem, N_DEV)
        for d in range(N_DEV):
            send_desc(d).start()

        scale = sx_ref[0] * sw_ref[0]

        for k in range(n_chunks):
            wcopy(k).wait()
            s, p = k % N_DEV, k // N_DEV
            w8_ref[pl.ds(s * k_per, k_per), :] = wf32_ref[k % NBUF].astype(
                jnp.float8_e5m2
            )
            if k + NBUF < n_chunks:
                wcopy(k + NBUF).start()
            if s == N_DEV - 1:
                if p == 0:
                    for src in range(N_DEV):
                        recv_desc(src).wait_recv()
                        a_ref[:, pl.ds(src * k_per, k_per)] = comm_ref[src]
                part = jnp.dot(
                    a_ref[...], w8_ref[...],
                    preferred_element_type=jnp.float32,
                )
                out_ref[:, pl.ds(p * N_TILE, N_TILE)] = part * scale

        for d in range(N_DEV):
            send_desc(d).wait_send()

    return pl.pallas_call(
        body,
        out_shape=jax.ShapeDtypeStruct((m_per, n), jnp.float32),
        in_specs=[
            pl.BlockSpec(memory_space=pltpu.VMEM),
            pl.BlockSpec(memory_space=pltpu.MemorySpace.HBM),
            pl.BlockSpec(memory_space=pltpu.SMEM),
            pl.BlockSpec(memory_space=pltpu.SMEM),
        ],
        out_specs=pl.BlockSpec(memory_space=pltpu.VMEM),
        scratch_shapes=[
            pltpu.VMEM((N_DEV, m_per, k_per), jnp.float8_e5m2),
            pltpu.VMEM((N_DEV, m_per, k_per), jnp.float8_e5m2),
            pltpu.VMEM((m_per, k_total), jnp.float8_e5m2),
            pltpu.VMEM((NBUF, k_per, N_TILE), jnp.float32),
            pltpu.VMEM((k_total, N_TILE), jnp.float8_e5m2),
            pltpu.SemaphoreType.DMA((N_DEV,)),
            pltpu.SemaphoreType.DMA((N_DEV,)),
            pltpu.SemaphoreType.DMA((4,)),
        ],
        compiler_params=pltpu.CompilerParams(
            collective_id=0,
            vmem_limit_bytes=100 * 1024 * 1024,
        ),
    )(x, w_mat, scale_x, scale_w)


# device time: 82443 ns/iter; 1.3655x vs baseline; 1.0064x over previous
import jax
import jax.numpy as jnp
from jax import lax
from jax.experimental import pallas as pl
from jax.experimental.pallas import tpu as pltpu

N_DEV = 8
N_TILE = 2048
NBUF = 3


def kernel(x, w_mat, scale_x, scale_w):
    m_total, k_per = x.shape
    k_total, n = w_mat.shape
    m_per = m_total // N_DEV
    n_panels = n // N_TILE
    n_chunks = n_panels * N_DEV

    def body(x_ref, w_hbm, sx_ref, sw_ref, out_ref,
             comm_ref, stage_ref, a_ref, wf32_ref, w8_ref, opan_ref,
             send_sems, recv_sems, wsems, osems):
        my = lax.axis_index("i")

        barrier_sem = pltpu.get_barrier_semaphore()
        for p in range(N_DEV):
            pl.semaphore_signal(
                barrier_sem, inc=1,
                device_id=(p,), device_id_type=pl.DeviceIdType.MESH,
            )

        def send_desc(d):
            return pltpu.make_async_remote_copy(
                src_ref=stage_ref.at[d],
                dst_ref=comm_ref.at[my],
                send_sem=send_sems.at[d],
                recv_sem=recv_sems.at[my],
                device_id=(d,),
                device_id_type=pl.DeviceIdType.MESH,
            )

        def recv_desc(src):
            return pltpu.make_async_remote_copy(
                src_ref=comm_ref.at[src],
                dst_ref=comm_ref.at[src],
                send_sem=send_sems.at[src],
                recv_sem=recv_sems.at[src],
                device_id=(src,),
                device_id_type=pl.DeviceIdType.MESH,
            )

        def wcopy(k):
            s, p = k % N_DEV, k // N_DEV
            return pltpu.make_async_copy(
                w_hbm.at[pl.ds(s * k_per, k_per), pl.ds(p * N_TILE, N_TILE)],
                wf32_ref.at[k % NBUF],
                wsems.at[k % 4],
            )

        for k in range(NBUF):
            wcopy(k).start()
        for d in range(N_DEV):
            stage_ref[d] = x_ref[pl.ds(d * m_per, m_per), :].astype(
                jnp.float8_e5m2
            )
        pl.semaphore_wait(barrier_sem, N_DEV)
        for d in range(N_DEV):
            send_desc(d).start()

        scale = sx_ref[0] * sw_ref[0]

        def ocopy(p):
            return pltpu.make_async_copy(
                opan_ref.at[p % 2],
                out_ref.at[:, pl.ds(p * N_TILE, N_TILE)],
                osems.at[p % 2],
            )

        for k in range(n_chunks):
            wcopy(k).wait()
            s, p = k % N_DEV, k // N_DEV
            w8_ref[pl.ds(s * k_per, k_per), :] = wf32_ref[k % NBUF].astype(
                jnp.float8_e5m2
            )
            if k + NBUF < n_chunks:
                wcopy(k + NBUF).start()
            if s == N_DEV - 1:
                if p == 0:
                    for src in range(N_DEV):
                        recv_desc(src).wait_recv()
                        a_ref[:, pl.ds(src * k_per, k_per)] = comm_ref[src]
                if p >= 2:
                    ocopy(p - 2).wait()
                part = jnp.dot(
                    a_ref[...], w8_ref[...],
                    preferred_element_type=jnp.float32,
                )
                opan_ref[p % 2] = part * scale
                ocopy(p).start()

        ocopy(n_panels - 2).wait()
        ocopy(n_panels - 1).wait()
        for d in range(N_DEV):
            send_desc(d).wait_send()

    return pl.pallas_call(
        body,
        out_shape=jax.ShapeDtypeStruct((m_per, n), jnp.float32),
        in_specs=[
            pl.BlockSpec(memory_space=pltpu.VMEM),
            pl.BlockSpec(memory_space=pltpu.MemorySpace.HBM),
            pl.BlockSpec(memory_space=pltpu.SMEM),
            pl.BlockSpec(memory_space=pltpu.SMEM),
        ],
        out_specs=pl.BlockSpec(memory_space=pltpu.MemorySpace.HBM),
        scratch_shapes=[
            pltpu.VMEM((N_DEV, m_per, k_per), jnp.float8_e5m2),
            pltpu.VMEM((N_DEV, m_per, k_per), jnp.float8_e5m2),
            pltpu.VMEM((m_per, k_total), jnp.float8_e5m2),
            pltpu.VMEM((NBUF, k_per, N_TILE), jnp.float32),
            pltpu.VMEM((k_total, N_TILE), jnp.float8_e5m2),
            pltpu.VMEM((2, m_per, N_TILE), jnp.float32),
            pltpu.SemaphoreType.DMA((N_DEV,)),
            pltpu.SemaphoreType.DMA((N_DEV,)),
            pltpu.SemaphoreType.DMA((4,)),
            pltpu.SemaphoreType.DMA((2,)),
        ],
        compiler_params=pltpu.CompilerParams(
            collective_id=0,
            vmem_limit_bytes=100 * 1024 * 1024,
        ),
    )(x, w_mat, scale_x, scale_w)
